# baseline (speedup 1.0000x reference)
import functools

import jax
import jax.numpy as jnp
from jax import lax
from jax.experimental import pallas as pl
from jax.experimental.pallas import tpu as pltpu
from jax.experimental.pallas import tpu_sc as plsc

B = 1024
N = 100000
K = 200
NC = 2
NS = 16
NW = NC * NS
ROWS_PER_W = B // NW
GROUPS_PER_W = ROWS_PER_W // 8

W = 6912
NFULL = N // W
TAILC = NFULL * W
TAILW = 3200
SIDEC = TAILC + TAILW
SIDEW = N - SIDEC
TAILX = TAILW + SIDEW
NCHUNK = -(-K // 16)
LASTO = K - 16


def _extract(buf_v, idx_v, out_v, r0, cb, w):
    lanes = lax.iota(jnp.int32, 16)
    cb_vec = jnp.full((16,), cb, jnp.int32)
    w_vec = jnp.full((16,), w, jnp.uint32)

    def row_body(s, _):
        r = r0 + s
        row_s = jnp.full((16,), s, jnp.int32)
        row_r = jnp.full((16,), r, jnp.int32)
        for c in range(NCHUNK):
            o = c * 16 if c < NCHUNK - 1 else LASTO
            rel = idx_v[r, pl.ds(o, 16)] - cb_vec
            m = plsc.bitcast(rel, jnp.uint32) < w_vec
            v = plsc.load_gather(buf_v, [row_s, rel], mask=m)
            ovec = jnp.full((16,), o, jnp.int32) + lanes
            plsc.store_scatter(out_v, [row_r, ovec], v, mask=m)
        return 0

    lax.fori_loop(0, 8, row_body, 0)


def _gather_body(inp_hbm, side_hbm, idx_hbm, out_hbm,
                 idx_v, buf_a, buf_b, side_v, out_v, sem_a, sem_b):
    wid = lax.axis_index("s") * NC + lax.axis_index("c")
    r_base = wid * ROWS_PER_W

    pltpu.sync_copy(idx_hbm.at[pl.ds(r_base, ROWS_PER_W)], idx_v)
    pltpu.sync_copy(side_hbm.at[pl.ds(r_base, ROWS_PER_W)], side_v)

    def group_body(g, _):
        rg = r_base + g * 8
        r0 = g * 8

        def dma(b, buf, sem):
            return pltpu.make_async_copy(
                inp_hbm.at[pl.ds(rg, 8), pl.ds(b * W, W)], buf, sem)

        dma(0, buf_a, sem_a).start()
        dma(1, buf_b, sem_b).start()

        def pair_body(bb, _):
            b0 = bb * 2
            dma(b0, buf_a, sem_a).wait()
            _extract(buf_a, idx_v, out_v, r0, b0 * W, W)
            dma(b0 + 2, buf_a, sem_a).start()
            dma(b0 + 1, buf_b, sem_b).wait()
            _extract(buf_b, idx_v, out_v, r0, (b0 + 1) * W, W)
            dma(b0 + 3, buf_b, sem_b).start()
            return 0

        lax.fori_loop(0, NFULL // 2 - 1, pair_body, 0)

        dma(NFULL - 2, buf_a, sem_a).wait()
        _extract(buf_a, idx_v, out_v, r0, (NFULL - 2) * W, W)
        dma(NFULL - 1, buf_b, sem_b).wait()
        _extract(buf_b, idx_v, out_v, r0, (NFULL - 1) * W, W)

        pltpu.sync_copy(
            inp_hbm.at[pl.ds(rg, 8), pl.ds(TAILC, TAILW)],
            buf_a.at[:, pl.ds(0, TAILW)])

        def fill_body(s, _):
            for c2 in range(SIDEW // 16):
                buf_a[s, pl.ds(TAILW + c2 * 16, 16)] = (
                    side_v[r0 + s, pl.ds(c2 * 16, 16)])
            return 0

        lax.fori_loop(0, 8, fill_body, 0)
        _extract(buf_a, idx_v, out_v, r0, TAILC, TAILX)
        return 0

    lax.fori_loop(0, GROUPS_PER_W, group_body, 0)

    pltpu.sync_copy(out_v, out_hbm.at[pl.ds(r_base, ROWS_PER_W)])


@jax.jit
def _gather(inp, side, idx):
    mesh = plsc.VectorSubcoreMesh(core_axis_name="c", subcore_axis_name="s")
    k = functools.partial(
        pl.kernel,
        mesh=mesh,
        out_type=jax.ShapeDtypeStruct((B, K), jnp.float32),
        scratch_types=[
            pltpu.VMEM((ROWS_PER_W, K), jnp.int32),
            pltpu.VMEM((8, W), jnp.float32),
            pltpu.VMEM((8, W), jnp.float32),
            pltpu.VMEM((ROWS_PER_W, SIDEW), jnp.float32),
            pltpu.VMEM((ROWS_PER_W, K), jnp.float32),
            pltpu.SemaphoreType.DMA,
            pltpu.SemaphoreType.DMA,
        ],
        compiler_params=pltpu.CompilerParams(
            disable_bounds_checks=True, needs_layout_passes=False,
            skip_device_barrier=True),
    )(_gather_body)
    return k(inp, side, idx)


def kernel(input, index):
    side = lax.slice(input, (0, SIDEC), (B, N))
    return _gather(input, side, index.astype(jnp.int32))
